# baseline (speedup 1.0000x reference)
import functools

import jax
import jax.numpy as jnp
from jax.experimental import pallas as pl
from jax.experimental.pallas import tpu as pltpu


def _gelu_exact(x):
    return 0.5 * x * (1.0 + jax.lax.erf(x * 0.7071067811865476))


def _gelu_tanh(x):
    return jax.nn.gelu(x, approximate=True)


def _silu(x):
    return x * jax.nn.sigmoid(x)


def _heads_body(x_ref, rW_ref, rb_ref, combo_W_ref, combo_b_ref, cW1_ref,
                cb1_ref, cW2_ref, cb2_ref, head_ref, cap_ref):
    x = x_ref[...]
    rv = _gelu_exact(
        jnp.dot(x, rW_ref[...], preferred_element_type=jnp.float32)
        + rb_ref[...])
    head_ref[...] = (
        jnp.dot(rv, combo_W_ref[...], preferred_element_type=jnp.float32)
        + combo_b_ref[...])
    c = _silu(
        jnp.dot(x, cW1_ref[...], preferred_element_type=jnp.float32)
        + cb1_ref[...])
    cap_ref[...] = (
        jnp.dot(c, cW2_ref[...], preferred_element_type=jnp.float32)
        + cb2_ref[...])


def _run_heads(xf, router_W, router_b, combo_W, combo_b, cap_W1, cap_b1,
               cap_W2p, cap_b2p, tile):
    n, d = xf.shape
    grid = (n // tile,)
    full = lambda a: pl.BlockSpec(a.shape, lambda t: (0,) * a.ndim)
    head, cap = pl.pallas_call(
        _heads_body,
        grid=grid,
        in_specs=[
            pl.BlockSpec((tile, d), lambda t: (t, 0)),
            full(router_W), full(router_b.reshape(1, -1)),
            full(combo_W), full(combo_b),
            full(cap_W1), full(cap_b1.reshape(1, -1)),
            full(cap_W2p), full(cap_b2p),
        ],
        out_specs=[
            pl.BlockSpec((tile, 128), lambda t: (t, 0)),
            pl.BlockSpec((tile, 128), lambda t: (t, 0)),
        ],
        out_shape=[
            jax.ShapeDtypeStruct((n, 128), jnp.float32),
            jax.ShapeDtypeStruct((n, 128), jnp.float32),
        ],
    )(xf, router_W, router_b.reshape(1, -1), combo_W, combo_b,
      cap_W1, cap_b1.reshape(1, -1), cap_W2p, cap_b2p)
    return head, cap


def _thresh_body(head_ref, bias_ref, mask_ref, wgt_ref, *, k, n_experts):
    logits = head_ref[...]
    scores = jax.nn.sigmoid(logits) + bias_ref[...]
    lane = jax.lax.broadcasted_iota(jnp.int32, scores.shape, 1)
    valid = lane < n_experts
    bits = jax.lax.bitcast_convert_type(scores, jnp.int32)
    ikey = jnp.where(bits >= 0, bits, bits ^ jnp.int32(0x7FFFFFFF))
    ikey = jnp.where(valid, ikey, jnp.int32(-2147483647 - 1))

    def body(i, res):
        bit = jnp.int32(1) << (jnp.int32(30) - i)
        cand = res + bit
        count = jnp.sum(jnp.where(valid & (ikey >= cand), 1, 0))
        return jnp.where(count >= k, cand, res)

    n_nonneg = jnp.sum(jnp.where(valid & (ikey >= 0), 1, 0))
    res0 = jnp.where(n_nonneg >= k, jnp.int32(0), jnp.int32(-2147483647 - 1))
    res = jax.lax.fori_loop(0, 31, body, res0)
    mask = valid & (ikey >= res)
    mask_ref[...] = mask.astype(jnp.float32)
    wgt_ref[...] = jnp.where(mask, logits, 0.0)


def _run_thresh(head, bias_pad, k, n_experts):
    n = head.shape[0]
    mask, wgt = pl.pallas_call(
        functools.partial(_thresh_body, k=k, n_experts=n_experts),
        in_specs=[
            pl.BlockSpec((n, 128), lambda: (0, 0)),
            pl.BlockSpec((1, 128), lambda: (0, 0)),
        ],
        out_specs=[
            pl.BlockSpec((n, 128), lambda: (0, 0)),
            pl.BlockSpec((n, 128), lambda: (0, 0)),
        ],
        out_shape=[
            jax.ShapeDtypeStruct((n, 128), jnp.float32),
            jax.ShapeDtypeStruct((n, 128), jnp.float32),
        ],
    )(head, bias_pad)
    return mask, wgt


def _expert_body(x_ref, w1_ref, b1_ref, w2_ref, b2_ref, wgt_ref, out_ref,
                 acc_ref, *, n_experts):
    e = pl.program_id(0)
    t = pl.program_id(1)
    x = x_ref[...]
    h = _gelu_tanh(
        jnp.dot(x, w1_ref[0], preferred_element_type=jnp.float32)
        + b1_ref[0])
    eo = (jnp.dot(h, w2_ref[0], preferred_element_type=jnp.float32)
          + b2_ref[0])
    contrib = eo * wgt_ref[0]
    tile = x.shape[0]
    sl = pl.ds(t * tile, tile)

    @pl.when(e == 0)
    def _():
        acc_ref[sl, :] = contrib

    @pl.when(e > 0)
    def _():
        acc_ref[sl, :] = acc_ref[sl, :] + contrib

    @pl.when(e == n_experts - 1)
    def _():
        out_ref[...] = acc_ref[sl, :]


def _run_experts(xf, exp_W1, exp_b1, exp_W2, exp_b2, wgtE, tile):
    n, d = xf.shape
    n_experts, _, h_dim = exp_W1.shape
    grid = (n_experts, n // tile)
    out = pl.pallas_call(
        functools.partial(_expert_body, n_experts=n_experts),
        grid=grid,
        in_specs=[
            pl.BlockSpec((tile, d), lambda e, t: (t, 0)),
            pl.BlockSpec((1, d, h_dim), lambda e, t: (e, 0, 0)),
            pl.BlockSpec((1, 1, h_dim), lambda e, t: (e, 0, 0)),
            pl.BlockSpec((1, h_dim, d), lambda e, t: (e, 0, 0)),
            pl.BlockSpec((1, 1, d), lambda e, t: (e, 0, 0)),
            pl.BlockSpec((1, tile, 1), lambda e, t: (e, t, 0)),
        ],
        out_specs=pl.BlockSpec((tile, d), lambda e, t: (t, 0)),
        out_shape=jax.ShapeDtypeStruct((n, d), jnp.float32),
        scratch_shapes=[pltpu.VMEM((n, d), jnp.float32)],
    )(xf, exp_W1, exp_b1.reshape(n_experts, 1, h_dim), exp_W2,
      exp_b2.reshape(n_experts, 1, d), wgtE)
    return out


def kernel(x, router_W, router_b, gate_W, gate_b, tgt_W, tgt_b, cap_W1,
           cap_b1, cap_W2, cap_b2, exp_W1, exp_b1, exp_W2, exp_b2, bias):
    b, l, d = x.shape
    n = b * l
    n_experts = gate_W.shape[1]
    n_plr = tgt_W.shape[1]
    k = n

    xf = x.reshape(n, d)

    combo_W = jnp.zeros((d, 128), jnp.float32)
    combo_W = combo_W.at[:, :n_experts].set(gate_W)
    combo_W = combo_W.at[:, 16:16 + n_plr].set(tgt_W)
    combo_b = jnp.zeros((1, 128), jnp.float32)
    combo_b = combo_b.at[0, :n_experts].set(gate_b)
    combo_b = combo_b.at[0, 16:16 + n_plr].set(tgt_b)
    cap_W2p = jnp.zeros((d, 128), jnp.float32).at[:, :n_experts].set(cap_W2)
    cap_b2p = jnp.zeros((1, 128), jnp.float32).at[0, :n_experts].set(cap_b2)
    bias_pad = jnp.zeros((1, 128), jnp.float32).at[0, :n_experts].set(bias)

    head, cap128 = _run_heads(xf, router_W, router_b, combo_W, combo_b,
                              cap_W1, cap_b1, cap_W2p, cap_b2p, tile=512)
    mask128, wgt128 = _run_thresh(head, bias_pad, k, n_experts)

    wgtE = jnp.transpose(wgt128[:, :n_experts])[:, :, None]
    out = _run_experts(xf, exp_W1, exp_b1, exp_W2, exp_b2, wgtE, tile=256)

    mE = jnp.transpose(mask128[:, :n_experts])
    cap = cap128[:, :n_experts].reshape(n_experts, -1)
    plr = head[:, 16:16 + n_plr].reshape(b, l, n_plr)
    return (out.reshape(b, l, d), mE, cap, plr)
